# baseline (speedup 1.0000x reference)
import functools

import jax
import jax.numpy as jnp
from jax import lax
from jax.experimental import pallas as pl
from jax.experimental.pallas import tpu as pltpu
from jax.experimental.pallas import tpu_sc as plsc

_NUM_ENT = 100000
_DIM = 128
_NUM_REL2 = 400
_BATCH = 16384

_NC = 2
_NS = 16
_NW = _NC * _NS
_BPW = _BATCH // _NW
_CHUNK = 128
_NCHUNK = _BPW // _CHUNK
_XROWS = 3128
_XCLAMP = _NUM_ENT - _XROWS


def _gather_body(emb_hbm, reltab_hbm, sub_hbm, rel_hbm,
                 sub_out, rel_out, x_out,
                 sub_idx_v, rel_idx_v, sub_buf, rel_a, rel_b,
                 sem_gs, sem_ws, sem_ga, sem_gb, sem_wa, sem_wb, sem_x):
    c = lax.axis_index("c")
    s = lax.axis_index("s")
    wid = s * _NC + c
    base = wid * _BPW
    irow = wid * _NCHUNK
    xbase = pl.multiple_of(jnp.minimum(wid * _XROWS, _XCLAMP), 8)

    xcp = pltpu.async_copy(emb_hbm.at[pl.ds(xbase, _XROWS)],
                           x_out.at[pl.ds(xbase, _XROWS)], sem_x)

    pltpu.sync_copy(sub_hbm.at[pl.ds(irow, _NCHUNK)], sub_idx_v)
    pltpu.sync_copy(rel_hbm.at[pl.ds(irow, _NCHUNK)], rel_idx_v)

    sub_cps = [
        pltpu.async_copy(emb_hbm.at[sub_idx_v.at[j]],
                         sub_buf.at[pl.ds(j * _CHUNK, _CHUNK)], sem_gs)
        for j in range(_NCHUNK)
    ]
    ga = pltpu.async_copy(reltab_hbm.at[rel_idx_v.at[0]], rel_a, sem_ga)

    ga.wait()
    wa = pltpu.async_copy(rel_a, rel_out.at[pl.ds(base, _CHUNK)], sem_wa)
    gb = pltpu.async_copy(reltab_hbm.at[rel_idx_v.at[1]], rel_b, sem_gb)

    for cp in sub_cps:
        cp.wait()
    ws = pltpu.async_copy(sub_buf, sub_out.at[pl.ds(base, _BPW)], sem_ws)

    gb.wait()
    wb = pltpu.async_copy(rel_b, rel_out.at[pl.ds(base + _CHUNK, _CHUNK)],
                          sem_wb)
    wa.wait()
    ga2 = pltpu.async_copy(reltab_hbm.at[rel_idx_v.at[2]], rel_a, sem_ga)
    ga2.wait()
    wa2 = pltpu.async_copy(rel_a,
                           rel_out.at[pl.ds(base + 2 * _CHUNK, _CHUNK)],
                           sem_wa)
    wb.wait()
    gb2 = pltpu.async_copy(reltab_hbm.at[rel_idx_v.at[3]], rel_b, sem_gb)
    gb2.wait()
    wb2 = pltpu.async_copy(rel_b,
                           rel_out.at[pl.ds(base + 3 * _CHUNK, _CHUNK)],
                           sem_wb)

    ws.wait()
    wa2.wait()
    wb2.wait()
    xcp.wait()


@functools.partial(
    pl.kernel,
    out_type=(
        jax.ShapeDtypeStruct((_BATCH, _DIM), jnp.float32),
        jax.ShapeDtypeStruct((_BATCH, _DIM), jnp.float32),
        jax.ShapeDtypeStruct((_NUM_ENT, _DIM), jnp.float32),
    ),
    mesh=plsc.VectorSubcoreMesh(core_axis_name="c", subcore_axis_name="s"),
    scratch_types=(
        pltpu.VMEM((_NCHUNK, _CHUNK), jnp.int32),
        pltpu.VMEM((_NCHUNK, _CHUNK), jnp.int32),
        pltpu.VMEM((_BPW, _DIM), jnp.float32),
        pltpu.VMEM((_CHUNK, _DIM), jnp.float32),
        pltpu.VMEM((_CHUNK, _DIM), jnp.float32),
        pltpu.SemaphoreType.DMA,
        pltpu.SemaphoreType.DMA,
        pltpu.SemaphoreType.DMA,
        pltpu.SemaphoreType.DMA,
        pltpu.SemaphoreType.DMA,
        pltpu.SemaphoreType.DMA,
        pltpu.SemaphoreType.DMA,
    ),
)
def _sc_gathers(emb_hbm, reltab_hbm, sub_hbm, rel_hbm,
                sub_out, rel_out, x_out,
                sub_idx_v, rel_idx_v, sub_buf, rel_a, rel_b,
                sem_gs, sem_ws, sem_ga, sem_gb, sem_wa, sem_wb, sem_x):
    _gather_body(emb_hbm, reltab_hbm, sub_hbm, rel_hbm,
                 sub_out, rel_out, x_out,
                 sub_idx_v, rel_idx_v, sub_buf, rel_a, rel_b,
                 sem_gs, sem_ws, sem_ga, sem_gb, sem_wa, sem_wb, sem_x)


def kernel(init_embed, init_rel, edge_index, edge_type, sub, rel):
    sub2 = sub.astype(jnp.int32).reshape(_BATCH // _CHUNK, _CHUNK)
    rel2 = rel.astype(jnp.int32).reshape(_BATCH // _CHUNK, _CHUNK)
    sub_emb, rel_emb, x_out = _sc_gathers(init_embed, init_rel, sub2, rel2)
    return (sub_emb, rel_emb, x_out)


# speedup vs baseline: 1.0026x; 1.0026x over previous
import functools

import jax
import jax.numpy as jnp
from jax import lax
from jax.experimental import pallas as pl
from jax.experimental.pallas import tpu as pltpu
from jax.experimental.pallas import tpu_sc as plsc

_NUM_ENT = 100000
_DIM = 128
_NUM_REL2 = 400
_BATCH = 16384

_NC = 2
_NS = 16
_NW = _NC * _NS
_BPW = _BATCH // _NW
_CHUNK = 128
_NCHUNK = _BPW // _CHUNK


def _gather_body(emb_hbm, reltab_hbm, sub_hbm, rel_hbm,
                 sub_out, rel_out,
                 sub_idx_v, rel_idx_v, sub_buf, rel_a, rel_b,
                 sem_gs, sem_ws, sem_ga, sem_gb, sem_wa, sem_wb):
    c = lax.axis_index("c")
    s = lax.axis_index("s")
    wid = s * _NC + c
    base = wid * _BPW
    irow = wid * _NCHUNK

    pltpu.sync_copy(sub_hbm.at[pl.ds(irow, _NCHUNK)], sub_idx_v)
    pltpu.sync_copy(rel_hbm.at[pl.ds(irow, _NCHUNK)], rel_idx_v)

    sub_cps = [
        pltpu.async_copy(emb_hbm.at[sub_idx_v.at[j]],
                         sub_buf.at[pl.ds(j * _CHUNK, _CHUNK)], sem_gs)
        for j in range(_NCHUNK)
    ]
    ga = pltpu.async_copy(reltab_hbm.at[rel_idx_v.at[0]], rel_a, sem_ga)

    ga.wait()
    wa = pltpu.async_copy(rel_a, rel_out.at[pl.ds(base, _CHUNK)], sem_wa)
    gb = pltpu.async_copy(reltab_hbm.at[rel_idx_v.at[1]], rel_b, sem_gb)

    for cp in sub_cps:
        cp.wait()
    ws = pltpu.async_copy(sub_buf, sub_out.at[pl.ds(base, _BPW)], sem_ws)

    gb.wait()
    wb = pltpu.async_copy(rel_b, rel_out.at[pl.ds(base + _CHUNK, _CHUNK)],
                          sem_wb)
    wa.wait()
    ga2 = pltpu.async_copy(reltab_hbm.at[rel_idx_v.at[2]], rel_a, sem_ga)
    ga2.wait()
    wa2 = pltpu.async_copy(rel_a,
                           rel_out.at[pl.ds(base + 2 * _CHUNK, _CHUNK)],
                           sem_wa)
    wb.wait()
    gb2 = pltpu.async_copy(reltab_hbm.at[rel_idx_v.at[3]], rel_b, sem_gb)
    gb2.wait()
    wb2 = pltpu.async_copy(rel_b,
                           rel_out.at[pl.ds(base + 3 * _CHUNK, _CHUNK)],
                           sem_wb)

    ws.wait()
    wa2.wait()
    wb2.wait()


@functools.partial(
    pl.kernel,
    out_type=(
        jax.ShapeDtypeStruct((_BATCH, _DIM), jnp.float32),
        jax.ShapeDtypeStruct((_BATCH, _DIM), jnp.float32),
    ),
    mesh=plsc.VectorSubcoreMesh(core_axis_name="c", subcore_axis_name="s"),
    scratch_types=(
        pltpu.VMEM((_NCHUNK, _CHUNK), jnp.int32),
        pltpu.VMEM((_NCHUNK, _CHUNK), jnp.int32),
        pltpu.VMEM((_BPW, _DIM), jnp.float32),
        pltpu.VMEM((_CHUNK, _DIM), jnp.float32),
        pltpu.VMEM((_CHUNK, _DIM), jnp.float32),
        pltpu.SemaphoreType.DMA,
        pltpu.SemaphoreType.DMA,
        pltpu.SemaphoreType.DMA,
        pltpu.SemaphoreType.DMA,
        pltpu.SemaphoreType.DMA,
        pltpu.SemaphoreType.DMA,
    ),
)
def _sc_gathers(emb_hbm, reltab_hbm, sub_hbm, rel_hbm, sub_out, rel_out,
                sub_idx_v, rel_idx_v, sub_buf, rel_a, rel_b,
                sem_gs, sem_ws, sem_ga, sem_gb, sem_wa, sem_wb):
    _gather_body(emb_hbm, reltab_hbm, sub_hbm, rel_hbm, sub_out, rel_out,
                 sub_idx_v, rel_idx_v, sub_buf, rel_a, rel_b,
                 sem_gs, sem_ws, sem_ga, sem_gb, sem_wa, sem_wb)


_NCOPY = 8
_COPY_ROWS = _NUM_ENT // _NCOPY


def _copy_body(x_ref, o_ref, *sems):
    cps = [
        pltpu.make_async_copy(
            x_ref.at[pl.ds(i * _COPY_ROWS, _COPY_ROWS)],
            o_ref.at[pl.ds(i * _COPY_ROWS, _COPY_ROWS)],
            sems[i],
        )
        for i in range(_NCOPY)
    ]
    for cp in cps:
        cp.start()
    for cp in cps:
        cp.wait()


_tc_copy = pl.pallas_call(
    _copy_body,
    out_shape=jax.ShapeDtypeStruct((_NUM_ENT, _DIM), jnp.float32),
    in_specs=[pl.BlockSpec(memory_space=pltpu.HBM)],
    out_specs=pl.BlockSpec(memory_space=pltpu.HBM),
    scratch_shapes=[pltpu.SemaphoreType.DMA] * _NCOPY,
)


def kernel(init_embed, init_rel, edge_index, edge_type, sub, rel):
    sub2 = sub.astype(jnp.int32).reshape(_BATCH // _CHUNK, _CHUNK)
    rel2 = rel.astype(jnp.int32).reshape(_BATCH // _CHUNK, _CHUNK)
    sub_emb, rel_emb = _sc_gathers(init_embed, init_rel, sub2, rel2)
    x_out = _tc_copy(init_embed)
    return (sub_emb, rel_emb, x_out)


# speedup vs baseline: 23.1019x; 23.0417x over previous
import functools

import jax
import jax.numpy as jnp
from jax import lax
from jax.experimental import pallas as pl
from jax.experimental.pallas import tpu as pltpu
from jax.experimental.pallas import tpu_sc as plsc

_NUM_ENT = 100000
_DIM = 128
_NUM_REL2 = 400
_BATCH = 16384

_NC = 2
_NS = 16
_NW = _NC * _NS
_BPW = _BATCH // _NW
_CHUNK = 128
_NCHUNK = _BPW // _CHUNK


def _gather_body(emb_hbm, reltab_hbm, sub_hbm, rel_hbm,
                 sub_out, rel_out,
                 sub_idx_v, rel_idx_v, sub_buf, rel_a, rel_b,
                 sem_gs, sem_ws, sem_ga, sem_gb, sem_wa, sem_wb):
    c = lax.axis_index("c")
    s = lax.axis_index("s")
    wid = s * _NC + c
    base = wid * _BPW
    irow = wid * _NCHUNK

    pltpu.sync_copy(sub_hbm.at[pl.ds(irow, _NCHUNK)], sub_idx_v)
    pltpu.sync_copy(rel_hbm.at[pl.ds(irow, _NCHUNK)], rel_idx_v)

    sub_cps = [
        pltpu.async_copy(emb_hbm.at[sub_idx_v.at[j]],
                         sub_buf.at[pl.ds(j * _CHUNK, _CHUNK)], sem_gs)
        for j in range(_NCHUNK)
    ]
    ga = pltpu.async_copy(reltab_hbm.at[rel_idx_v.at[0]], rel_a, sem_ga)

    ga.wait()
    wa = pltpu.async_copy(rel_a, rel_out.at[pl.ds(base, _CHUNK)], sem_wa)
    gb = pltpu.async_copy(reltab_hbm.at[rel_idx_v.at[1]], rel_b, sem_gb)

    for cp in sub_cps:
        cp.wait()
    ws = pltpu.async_copy(sub_buf, sub_out.at[pl.ds(base, _BPW)], sem_ws)

    gb.wait()
    wb = pltpu.async_copy(rel_b, rel_out.at[pl.ds(base + _CHUNK, _CHUNK)],
                          sem_wb)
    wa.wait()
    ga2 = pltpu.async_copy(reltab_hbm.at[rel_idx_v.at[2]], rel_a, sem_ga)
    ga2.wait()
    wa2 = pltpu.async_copy(rel_a,
                           rel_out.at[pl.ds(base + 2 * _CHUNK, _CHUNK)],
                           sem_wa)
    wb.wait()
    gb2 = pltpu.async_copy(reltab_hbm.at[rel_idx_v.at[3]], rel_b, sem_gb)
    gb2.wait()
    wb2 = pltpu.async_copy(rel_b,
                           rel_out.at[pl.ds(base + 3 * _CHUNK, _CHUNK)],
                           sem_wb)

    ws.wait()
    wa2.wait()
    wb2.wait()


@functools.partial(
    pl.kernel,
    out_type=(
        jax.ShapeDtypeStruct((_BATCH, _DIM), jnp.float32),
        jax.ShapeDtypeStruct((_BATCH, _DIM), jnp.float32),
    ),
    mesh=plsc.VectorSubcoreMesh(core_axis_name="c", subcore_axis_name="s"),
    scratch_types=(
        pltpu.VMEM((_NCHUNK, _CHUNK), jnp.int32),
        pltpu.VMEM((_NCHUNK, _CHUNK), jnp.int32),
        pltpu.VMEM((_BPW, _DIM), jnp.float32),
        pltpu.VMEM((_CHUNK, _DIM), jnp.float32),
        pltpu.VMEM((_CHUNK, _DIM), jnp.float32),
        pltpu.SemaphoreType.DMA,
        pltpu.SemaphoreType.DMA,
        pltpu.SemaphoreType.DMA,
        pltpu.SemaphoreType.DMA,
        pltpu.SemaphoreType.DMA,
        pltpu.SemaphoreType.DMA,
    ),
)
def _sc_gathers(emb_hbm, reltab_hbm, sub_hbm, rel_hbm, sub_out, rel_out,
                sub_idx_v, rel_idx_v, sub_buf, rel_a, rel_b,
                sem_gs, sem_ws, sem_ga, sem_gb, sem_wa, sem_wb):
    _gather_body(emb_hbm, reltab_hbm, sub_hbm, rel_hbm, sub_out, rel_out,
                 sub_idx_v, rel_idx_v, sub_buf, rel_a, rel_b,
                 sem_gs, sem_ws, sem_ga, sem_gb, sem_wa, sem_wb)


_COPY_ROWS = 5000


def _copy_body(x_ref, o_ref):
    o_ref[...] = x_ref[...]


_tc_copy = pl.pallas_call(
    _copy_body,
    out_shape=jax.ShapeDtypeStruct((_NUM_ENT, _DIM), jnp.float32),
    grid=(_NUM_ENT // _COPY_ROWS,),
    in_specs=[pl.BlockSpec((_COPY_ROWS, _DIM), lambda i: (i, 0))],
    out_specs=pl.BlockSpec((_COPY_ROWS, _DIM), lambda i: (i, 0)),
)


def kernel(init_embed, init_rel, edge_index, edge_type, sub, rel):
    sub2 = sub.astype(jnp.int32).reshape(_BATCH // _CHUNK, _CHUNK)
    rel2 = rel.astype(jnp.int32).reshape(_BATCH // _CHUNK, _CHUNK)
    sub_emb, rel_emb = _sc_gathers(init_embed, init_rel, sub2, rel2)
    x_out = _tc_copy(init_embed)
    return (sub_emb, rel_emb, x_out)


# speedup vs baseline: 23.6938x; 1.0256x over previous
import functools

import jax
import jax.numpy as jnp
from jax import lax
from jax.experimental import pallas as pl
from jax.experimental.pallas import tpu as pltpu
from jax.experimental.pallas import tpu_sc as plsc

_NUM_ENT = 100000
_DIM = 128
_NUM_REL2 = 400
_BATCH = 16384

_NC = 2
_NS = 16
_NW = _NC * _NS
_BPW = _BATCH // _NW
_CHUNK = 128
_NCHUNK = _BPW // _CHUNK


def _gather_body(emb_hbm, reltab_hbm, sub_hbm, rel_hbm,
                 sub_out, rel_out,
                 sub_idx_v, rel_idx_v, sub_buf, rel_a, rel_b,
                 sem_gs, sem_ws, sem_ga, sem_gb, sem_wa, sem_wb):
    c = lax.axis_index("c")
    s = lax.axis_index("s")
    wid = s * _NC + c
    base = wid * _BPW
    irow = wid * _NCHUNK

    pltpu.sync_copy(sub_hbm.at[pl.ds(irow, _NCHUNK)], sub_idx_v)
    pltpu.sync_copy(rel_hbm.at[pl.ds(irow, _NCHUNK)], rel_idx_v)

    sub_cps = [
        pltpu.async_copy(emb_hbm.at[sub_idx_v.at[j]],
                         sub_buf.at[pl.ds(j * _CHUNK, _CHUNK)], sem_gs)
        for j in range(_NCHUNK)
    ]
    ga = pltpu.async_copy(reltab_hbm.at[rel_idx_v.at[0]], rel_a, sem_ga)

    ga.wait()
    wa = pltpu.async_copy(rel_a, rel_out.at[pl.ds(base, _CHUNK)], sem_wa)
    gb = pltpu.async_copy(reltab_hbm.at[rel_idx_v.at[1]], rel_b, sem_gb)

    for cp in sub_cps:
        cp.wait()
    ws = pltpu.async_copy(sub_buf, sub_out.at[pl.ds(base, _BPW)], sem_ws)

    gb.wait()
    wb = pltpu.async_copy(rel_b, rel_out.at[pl.ds(base + _CHUNK, _CHUNK)],
                          sem_wb)
    wa.wait()
    ga2 = pltpu.async_copy(reltab_hbm.at[rel_idx_v.at[2]], rel_a, sem_ga)
    ga2.wait()
    wa2 = pltpu.async_copy(rel_a,
                           rel_out.at[pl.ds(base + 2 * _CHUNK, _CHUNK)],
                           sem_wa)
    wb.wait()
    gb2 = pltpu.async_copy(reltab_hbm.at[rel_idx_v.at[3]], rel_b, sem_gb)
    gb2.wait()
    wb2 = pltpu.async_copy(rel_b,
                           rel_out.at[pl.ds(base + 3 * _CHUNK, _CHUNK)],
                           sem_wb)

    ws.wait()
    wa2.wait()
    wb2.wait()


@functools.partial(
    pl.kernel,
    out_type=(
        jax.ShapeDtypeStruct((_BATCH, _DIM), jnp.float32),
        jax.ShapeDtypeStruct((_BATCH, _DIM), jnp.float32),
    ),
    mesh=plsc.VectorSubcoreMesh(core_axis_name="c", subcore_axis_name="s"),
    scratch_types=(
        pltpu.VMEM((_NCHUNK, _CHUNK), jnp.int32),
        pltpu.VMEM((_NCHUNK, _CHUNK), jnp.int32),
        pltpu.VMEM((_BPW, _DIM), jnp.float32),
        pltpu.VMEM((_CHUNK, _DIM), jnp.float32),
        pltpu.VMEM((_CHUNK, _DIM), jnp.float32),
        pltpu.SemaphoreType.DMA,
        pltpu.SemaphoreType.DMA,
        pltpu.SemaphoreType.DMA,
        pltpu.SemaphoreType.DMA,
        pltpu.SemaphoreType.DMA,
        pltpu.SemaphoreType.DMA,
    ),
)
def _sc_gathers(emb_hbm, reltab_hbm, sub_hbm, rel_hbm, sub_out, rel_out,
                sub_idx_v, rel_idx_v, sub_buf, rel_a, rel_b,
                sem_gs, sem_ws, sem_ga, sem_gb, sem_wa, sem_wb):
    _gather_body(emb_hbm, reltab_hbm, sub_hbm, rel_hbm, sub_out, rel_out,
                 sub_idx_v, rel_idx_v, sub_buf, rel_a, rel_b,
                 sem_gs, sem_ws, sem_ga, sem_gb, sem_wa, sem_wb)


_COPY_ROWS = 10000


def _copy_body(x_ref, o_ref):
    o_ref[...] = x_ref[...]


_tc_copy = pl.pallas_call(
    _copy_body,
    out_shape=jax.ShapeDtypeStruct((_NUM_ENT, _DIM), jnp.float32),
    grid=(_NUM_ENT // _COPY_ROWS,),
    in_specs=[pl.BlockSpec((_COPY_ROWS, _DIM), lambda i: (i, 0))],
    out_specs=pl.BlockSpec((_COPY_ROWS, _DIM), lambda i: (i, 0)),
)


def kernel(init_embed, init_rel, edge_index, edge_type, sub, rel):
    sub2 = sub.astype(jnp.int32).reshape(_BATCH // _CHUNK, _CHUNK)
    rel2 = rel.astype(jnp.int32).reshape(_BATCH // _CHUNK, _CHUNK)
    sub_emb, rel_emb = _sc_gathers(init_embed, init_rel, sub2, rel2)
    x_out = _tc_copy(init_embed)
    return (sub_emb, rel_emb, x_out)


# speedup vs baseline: 23.7034x; 1.0004x over previous
import functools

import jax
import jax.numpy as jnp
from jax import lax
from jax.experimental import pallas as pl
from jax.experimental.pallas import tpu as pltpu
from jax.experimental.pallas import tpu_sc as plsc

_NUM_ENT = 100000
_DIM = 128
_NUM_REL2 = 400
_BATCH = 16384

_NC = 2
_NS = 16
_NW = _NC * _NS
_BPW = _BATCH // _NW
_CHUNK = 128
_NCHUNK = _BPW // _CHUNK


def _gather_body(emb_hbm, reltab_hbm, sub_hbm, rel_hbm,
                 sub_out, rel_out,
                 sub_idx_v, rel_idx_v, sub_buf, rel_a, rel_b,
                 sem_gs, sem_ws, sem_ga, sem_gb, sem_wa, sem_wb):
    c = lax.axis_index("c")
    s = lax.axis_index("s")
    wid = s * _NC + c
    base = wid * _BPW
    irow = wid * _NCHUNK

    pltpu.sync_copy(sub_hbm.at[pl.ds(irow, _NCHUNK)], sub_idx_v)
    pltpu.sync_copy(rel_hbm.at[pl.ds(irow, _NCHUNK)], rel_idx_v)

    sub_cps = [
        pltpu.async_copy(emb_hbm.at[sub_idx_v.at[j]],
                         sub_buf.at[pl.ds(j * _CHUNK, _CHUNK)], sem_gs)
        for j in range(_NCHUNK)
    ]
    ga = pltpu.async_copy(reltab_hbm.at[rel_idx_v.at[0]], rel_a, sem_ga)

    ga.wait()
    wa = pltpu.async_copy(rel_a, rel_out.at[pl.ds(base, _CHUNK)], sem_wa)
    gb = pltpu.async_copy(reltab_hbm.at[rel_idx_v.at[1]], rel_b, sem_gb)

    for cp in sub_cps:
        cp.wait()
    ws = pltpu.async_copy(sub_buf, sub_out.at[pl.ds(base, _BPW)], sem_ws)

    gb.wait()
    wb = pltpu.async_copy(rel_b, rel_out.at[pl.ds(base + _CHUNK, _CHUNK)],
                          sem_wb)
    wa.wait()
    ga2 = pltpu.async_copy(reltab_hbm.at[rel_idx_v.at[2]], rel_a, sem_ga)
    ga2.wait()
    wa2 = pltpu.async_copy(rel_a,
                           rel_out.at[pl.ds(base + 2 * _CHUNK, _CHUNK)],
                           sem_wa)
    wb.wait()
    gb2 = pltpu.async_copy(reltab_hbm.at[rel_idx_v.at[3]], rel_b, sem_gb)
    gb2.wait()
    wb2 = pltpu.async_copy(rel_b,
                           rel_out.at[pl.ds(base + 3 * _CHUNK, _CHUNK)],
                           sem_wb)

    ws.wait()
    wa2.wait()
    wb2.wait()


@functools.partial(
    pl.kernel,
    out_type=(
        jax.ShapeDtypeStruct((_BATCH, _DIM), jnp.float32),
        jax.ShapeDtypeStruct((_BATCH, _DIM), jnp.float32),
    ),
    mesh=plsc.VectorSubcoreMesh(core_axis_name="c", subcore_axis_name="s"),
    scratch_types=(
        pltpu.VMEM((_NCHUNK, _CHUNK), jnp.int32),
        pltpu.VMEM((_NCHUNK, _CHUNK), jnp.int32),
        pltpu.VMEM((_BPW, _DIM), jnp.float32),
        pltpu.VMEM((_CHUNK, _DIM), jnp.float32),
        pltpu.VMEM((_CHUNK, _DIM), jnp.float32),
        pltpu.SemaphoreType.DMA,
        pltpu.SemaphoreType.DMA,
        pltpu.SemaphoreType.DMA,
        pltpu.SemaphoreType.DMA,
        pltpu.SemaphoreType.DMA,
        pltpu.SemaphoreType.DMA,
    ),
)
def _sc_gathers(emb_hbm, reltab_hbm, sub_hbm, rel_hbm, sub_out, rel_out,
                sub_idx_v, rel_idx_v, sub_buf, rel_a, rel_b,
                sem_gs, sem_ws, sem_ga, sem_gb, sem_wa, sem_wb):
    _gather_body(emb_hbm, reltab_hbm, sub_hbm, rel_hbm, sub_out, rel_out,
                 sub_idx_v, rel_idx_v, sub_buf, rel_a, rel_b,
                 sem_gs, sem_ws, sem_ga, sem_gb, sem_wa, sem_wb)


_COPY_ROWS = 20000


def _copy_body(x_ref, o_ref):
    o_ref[...] = x_ref[...]


_tc_copy = pl.pallas_call(
    _copy_body,
    out_shape=jax.ShapeDtypeStruct((_NUM_ENT, _DIM), jnp.float32),
    grid=(_NUM_ENT // _COPY_ROWS,),
    in_specs=[pl.BlockSpec((_COPY_ROWS, _DIM), lambda i: (i, 0))],
    out_specs=pl.BlockSpec((_COPY_ROWS, _DIM), lambda i: (i, 0)),
)


def kernel(init_embed, init_rel, edge_index, edge_type, sub, rel):
    sub2 = sub.astype(jnp.int32).reshape(_BATCH // _CHUNK, _CHUNK)
    rel2 = rel.astype(jnp.int32).reshape(_BATCH // _CHUNK, _CHUNK)
    sub_emb, rel_emb = _sc_gathers(init_embed, init_rel, sub2, rel2)
    x_out = _tc_copy(init_embed)
    return (sub_emb, rel_emb, x_out)


# speedup vs baseline: 24.3553x; 1.0275x over previous
import functools

import jax
import jax.numpy as jnp
from jax import lax
from jax.experimental import pallas as pl
from jax.experimental.pallas import tpu as pltpu
from jax.experimental.pallas import tpu_sc as plsc

_NUM_ENT = 100000
_DIM = 128
_NUM_REL2 = 400
_BATCH = 16384

_NC = 2
_NS = 16
_NW = _NC * _NS
_BPW = _BATCH // _NW
_CHUNK = 128
_NCHUNK = _BPW // _CHUNK


def _gather_body(emb_hbm, reltab_hbm, idx_hbm,
                 sub_out, rel_out,
                 idx_v, sub_buf, rel_a, rel_b,
                 sem_gs, sem_ws, sem_ga, sem_gb, sem_wa, sem_wb):
    c = lax.axis_index("c")
    s = lax.axis_index("s")
    wid = s * _NC + c
    base = wid * _BPW

    pltpu.sync_copy(idx_hbm.at[wid], idx_v)

    sub_cps = [
        pltpu.async_copy(emb_hbm.at[idx_v.at[j]],
                         sub_buf.at[pl.ds(j * _CHUNK, _CHUNK)], sem_gs)
        for j in range(_NCHUNK)
    ]
    g0 = pltpu.async_copy(reltab_hbm.at[idx_v.at[4]], rel_a, sem_ga)
    g1 = pltpu.async_copy(reltab_hbm.at[idx_v.at[5]], rel_b, sem_gb)

    g0.wait()
    w0 = pltpu.async_copy(rel_a, rel_out.at[pl.ds(base, _CHUNK)], sem_wa)
    g1.wait()
    w1 = pltpu.async_copy(rel_b, rel_out.at[pl.ds(base + _CHUNK, _CHUNK)],
                          sem_wb)

    for cp in sub_cps:
        cp.wait()
    ws = pltpu.async_copy(sub_buf, sub_out.at[pl.ds(base, _BPW)], sem_ws)

    w0.wait()
    g2 = pltpu.async_copy(reltab_hbm.at[idx_v.at[6]], rel_a, sem_ga)
    w1.wait()
    g3 = pltpu.async_copy(reltab_hbm.at[idx_v.at[7]], rel_b, sem_gb)
    g2.wait()
    w2 = pltpu.async_copy(rel_a,
                          rel_out.at[pl.ds(base + 2 * _CHUNK, _CHUNK)],
                          sem_wa)
    g3.wait()
    w3 = pltpu.async_copy(rel_b,
                          rel_out.at[pl.ds(base + 3 * _CHUNK, _CHUNK)],
                          sem_wb)

    ws.wait()
    w2.wait()
    w3.wait()


@functools.partial(
    pl.kernel,
    out_type=(
        jax.ShapeDtypeStruct((_BATCH, _DIM), jnp.float32),
        jax.ShapeDtypeStruct((_BATCH, _DIM), jnp.float32),
    ),
    mesh=plsc.VectorSubcoreMesh(core_axis_name="c", subcore_axis_name="s"),
    scratch_types=(
        pltpu.VMEM((2 * _NCHUNK, _CHUNK), jnp.int32),
        pltpu.VMEM((_BPW, _DIM), jnp.float32),
        pltpu.VMEM((_CHUNK, _DIM), jnp.float32),
        pltpu.VMEM((_CHUNK, _DIM), jnp.float32),
        pltpu.SemaphoreType.DMA,
        pltpu.SemaphoreType.DMA,
        pltpu.SemaphoreType.DMA,
        pltpu.SemaphoreType.DMA,
        pltpu.SemaphoreType.DMA,
        pltpu.SemaphoreType.DMA,
    ),
)
def _sc_gathers(emb_hbm, reltab_hbm, idx_hbm, sub_out, rel_out,
                idx_v, sub_buf, rel_a, rel_b,
                sem_gs, sem_ws, sem_ga, sem_gb, sem_wa, sem_wb):
    _gather_body(emb_hbm, reltab_hbm, idx_hbm, sub_out, rel_out,
                 idx_v, sub_buf, rel_a, rel_b,
                 sem_gs, sem_ws, sem_ga, sem_gb, sem_wa, sem_wb)


_COPY_ROWS = 20000


def _copy_body(x_ref, o_ref):
    o_ref[...] = x_ref[...]


_tc_copy = pl.pallas_call(
    _copy_body,
    out_shape=jax.ShapeDtypeStruct((_NUM_ENT, _DIM), jnp.float32),
    grid=(_NUM_ENT // _COPY_ROWS,),
    in_specs=[pl.BlockSpec((_COPY_ROWS, _DIM), lambda i: (i, 0))],
    out_specs=pl.BlockSpec((_COPY_ROWS, _DIM), lambda i: (i, 0)),
)


def kernel(init_embed, init_rel, edge_index, edge_type, sub, rel):
    sub3 = sub.astype(jnp.int32).reshape(_NW, _NCHUNK, _CHUNK)
    rel3 = rel.astype(jnp.int32).reshape(_NW, _NCHUNK, _CHUNK)
    idx_all = jnp.concatenate([sub3, rel3], axis=1)
    sub_emb, rel_emb = _sc_gathers(init_embed, init_rel, idx_all)
    x_out = _tc_copy(init_embed)
    return (sub_emb, rel_emb, x_out)


# speedup vs baseline: 24.3614x; 1.0002x over previous
import functools

import jax
import jax.numpy as jnp
from jax import lax
from jax.experimental import pallas as pl
from jax.experimental.pallas import tpu as pltpu
from jax.experimental.pallas import tpu_sc as plsc

_NUM_ENT = 100000
_DIM = 128
_NUM_REL2 = 400
_BATCH = 16384

_NC = 2
_NS = 16
_NW = _NC * _NS
_BPW = _BATCH // _NW
_CHUNK = 128
_NCHUNK = _BPW // _CHUNK


def _gather_body(emb_hbm, reltab_hbm, idx_hbm,
                 sub_out, rel_out,
                 idx_v, sub_buf, rel_a, rel_b,
                 sem_gs, sem_ws, sem_ga, sem_gb, sem_wa, sem_wb):
    c = lax.axis_index("c")
    s = lax.axis_index("s")
    wid = s * _NC + c
    base = wid * _BPW

    pltpu.sync_copy(idx_hbm.at[wid], idx_v)

    sub_cps = [
        pltpu.async_copy(emb_hbm.at[idx_v.at[j]],
                         sub_buf.at[pl.ds(j * _CHUNK, _CHUNK)], sem_gs)
        for j in range(_NCHUNK)
    ]
    g0 = pltpu.async_copy(reltab_hbm.at[idx_v.at[4]], rel_a, sem_ga)
    g1 = pltpu.async_copy(reltab_hbm.at[idx_v.at[5]], rel_b, sem_gb)

    g0.wait()
    w0 = pltpu.async_copy(rel_a, rel_out.at[pl.ds(base, _CHUNK)], sem_wa)
    g1.wait()
    w1 = pltpu.async_copy(rel_b, rel_out.at[pl.ds(base + _CHUNK, _CHUNK)],
                          sem_wb)

    for cp in sub_cps:
        cp.wait()
    ws = pltpu.async_copy(sub_buf, sub_out.at[pl.ds(base, _BPW)], sem_ws)

    w0.wait()
    g2 = pltpu.async_copy(reltab_hbm.at[idx_v.at[6]], rel_a, sem_ga)
    w1.wait()
    g3 = pltpu.async_copy(reltab_hbm.at[idx_v.at[7]], rel_b, sem_gb)
    g2.wait()
    w2 = pltpu.async_copy(rel_a,
                          rel_out.at[pl.ds(base + 2 * _CHUNK, _CHUNK)],
                          sem_wa)
    g3.wait()
    w3 = pltpu.async_copy(rel_b,
                          rel_out.at[pl.ds(base + 3 * _CHUNK, _CHUNK)],
                          sem_wb)

    ws.wait()
    w2.wait()
    w3.wait()


@functools.partial(
    pl.kernel,
    out_type=(
        jax.ShapeDtypeStruct((_BATCH, _DIM), jnp.float32),
        jax.ShapeDtypeStruct((_BATCH, _DIM), jnp.float32),
    ),
    mesh=plsc.VectorSubcoreMesh(core_axis_name="c", subcore_axis_name="s"),
    scratch_types=(
        pltpu.VMEM((2 * _NCHUNK, _CHUNK), jnp.int32),
        pltpu.VMEM((_BPW, _DIM), jnp.float32),
        pltpu.VMEM((_CHUNK, _DIM), jnp.float32),
        pltpu.VMEM((_CHUNK, _DIM), jnp.float32),
        pltpu.SemaphoreType.DMA,
        pltpu.SemaphoreType.DMA,
        pltpu.SemaphoreType.DMA,
        pltpu.SemaphoreType.DMA,
        pltpu.SemaphoreType.DMA,
        pltpu.SemaphoreType.DMA,
    ),
)
def _sc_gathers(emb_hbm, reltab_hbm, idx_hbm, sub_out, rel_out,
                idx_v, sub_buf, rel_a, rel_b,
                sem_gs, sem_ws, sem_ga, sem_gb, sem_wa, sem_wb):
    _gather_body(emb_hbm, reltab_hbm, idx_hbm, sub_out, rel_out,
                 idx_v, sub_buf, rel_a, rel_b,
                 sem_gs, sem_ws, sem_ga, sem_gb, sem_wa, sem_wb)


_COPY_ROWS = 20000


def _copy_body(x_ref, o_ref):
    o_ref[...] = x_ref[...]


_tc_copy = pl.pallas_call(
    _copy_body,
    out_shape=jax.ShapeDtypeStruct((_NUM_ENT, _DIM), jnp.float32),
    grid=(_NUM_ENT // _COPY_ROWS,),
    in_specs=[pl.BlockSpec((_COPY_ROWS, _DIM), lambda i: (i, 0))],
    out_specs=pl.BlockSpec((_COPY_ROWS, _DIM), lambda i: (i, 0)),
)


def kernel(init_embed, init_rel, edge_index, edge_type, sub, rel):
    sub3 = sub.astype(jnp.int32).reshape(_NW, _NCHUNK, _CHUNK)
    rel3 = rel.astype(jnp.int32).reshape(_NW, _NCHUNK, _CHUNK)
    idx_all = jnp.concatenate([sub3, rel3], axis=1)
    x_out = _tc_copy(init_embed)
    sub_emb, rel_emb = _sc_gathers(init_embed, init_rel, idx_all)
    return (sub_emb, rel_emb, x_out)
